# baseline (speedup 1.0000x reference)
import functools

import jax
import jax.numpy as jnp
from jax import lax
from jax.experimental import pallas as pl
from jax.experimental.pallas import tpu as pltpu
from jax.experimental.pallas import tpu_sc as plsc

_T = 64
_K = 3
_NW = 32


def _sc_row_gather(table, idx):
    m = idx.shape[0]
    h = table.shape[1]
    per_w = m // _NW
    ch = 48
    nch = per_w // ch
    assert per_w % ch == 0
    mesh = plsc.VectorSubcoreMesh(core_axis_name="c", subcore_axis_name="s")

    @functools.partial(
        pl.kernel,
        out_type=jax.ShapeDtypeStruct((m, h), jnp.float32),
        mesh=mesh,
        scratch_types=[
            pltpu.VMEM((ch,), jnp.int32),
            pltpu.VMEM((ch,), jnp.int32),
            pltpu.VMEM((ch, h), jnp.float32),
            pltpu.VMEM((ch, h), jnp.float32),
            pltpu.SemaphoreType.DMA,
            pltpu.SemaphoreType.DMA,
        ],
    )
    def k(table_hbm, idx_hbm, out_hbm, idx0, idx1, rows0, rows1, sem0, sem1):
        wid = lax.axis_index("s") * 2 + lax.axis_index("c")
        base = wid * per_w
        idxv, rowsv, sems = [idx0, idx1], [rows0, rows1], [sem0, sem1]
        handles = [None, None]
        pltpu.sync_copy(idx_hbm.at[pl.ds(base, ch)], idx0)
        handles[0] = pltpu.async_copy(table_hbm.at[idx0], rows0, sem0)
        for c in range(nch):
            j, jn = c % 2, (c + 1) % 2
            if c + 1 < nch:
                off = base + (c + 1) * ch
                pltpu.sync_copy(idx_hbm.at[pl.ds(off, ch)], idxv[jn])
                handles[jn] = pltpu.async_copy(
                    table_hbm.at[idxv[jn]], rowsv[jn], sems[jn])
            handles[j].wait()
            pltpu.sync_copy(rowsv[j], out_hbm.at[pl.ds(base + c * ch, ch)])

    return k(table, idx)


def _router_kernel(x_ref, gwu_ref, gbu_ref, probs_ref, misc_ref,
                   cnt_ref, ww_ref):
    i = pl.program_id(0)

    @pl.when(i == 0)
    def _():
        cnt_ref[...] = jnp.zeros_like(cnt_ref)
        ww_ref[...] = jnp.zeros_like(ww_ref)

    x = x_ref[...]
    la = jnp.dot(x, gwu_ref[...], preferred_element_type=jnp.float32) \
        + gbu_ref[...]
    ii = lax.broadcasted_iota(jnp.int32, la.shape, 1)
    ml = jnp.where(ii < 12, la, -1e30)
    probs = jax.nn.softmax(ml, axis=-1)
    probs_ref[...] = probs
    urg = jax.nn.sigmoid(jnp.sum(jnp.where(ii == 12, la, 0.0), axis=1,
                                 keepdims=True))

    pr = probs
    ams, vs, hs = [], [], []
    for _ in range(_K):
        m = jnp.max(pr, axis=1, keepdims=True)
        am = jnp.min(jnp.where(pr == m, ii, 128), axis=1, keepdims=True)
        h = (ii == am).astype(jnp.float32)
        pr = jnp.where(ii == am, -1.0, pr)
        ams.append(am)
        vs.append(m)
        hs.append(h)
    e1 = jnp.exp(vs[1] - vs[0])
    e2 = jnp.exp(vs[2] - vs[0])
    den = 1.0 + e1 + e2
    ews = [1.0 / den, e1 / den, e2 / den]

    c = hs[0] + hs[1] + hs[2]
    ri = lax.broadcasted_iota(jnp.int32, (c.shape[0], c.shape[0]), 0)
    ci = lax.broadcasted_iota(jnp.int32, (c.shape[0], c.shape[0]), 1)
    ltri = (ri > ci).astype(jnp.float32)
    base = jnp.dot(ltri, c, preferred_element_type=jnp.float32)
    carry = cnt_ref[...]
    misc = urg * (ii == 0).astype(jnp.float32)
    for k in range(_K):
        rank = jnp.sum((base + carry) * hs[k], axis=1, keepdims=True)
        misc = misc + ews[k] * (ii == 1 + k).astype(jnp.float32)
        misc = misc + ams[k].astype(jnp.float32) * (ii == 4 + k).astype(jnp.float32)
        misc = misc + rank * (ii == 7 + k).astype(jnp.float32)
    misc_ref[...] = misc
    cnt_ref[...] += jnp.sum(c, axis=0, keepdims=True)
    ww_ref[...] += jnp.sum(ews[0] * hs[0] + ews[1] * hs[1] + ews[2] * hs[2],
                           axis=0, keepdims=True)


def _run_router(tok, gwu, gbu, n, h):
    ts = 512
    return pl.pallas_call(
        _router_kernel,
        grid=(n // ts,),
        in_specs=[
            pl.BlockSpec((ts, h), lambda i: (i, 0)),
            pl.BlockSpec((h, 128), lambda i: (0, 0)),
            pl.BlockSpec((1, 128), lambda i: (0, 0)),
        ],
        out_specs=[
            pl.BlockSpec((ts, 128), lambda i: (i, 0)),
            pl.BlockSpec((ts, 128), lambda i: (i, 0)),
            pl.BlockSpec((1, 128), lambda i: (0, 0)),
            pl.BlockSpec((1, 128), lambda i: (0, 0)),
        ],
        out_shape=[
            jax.ShapeDtypeStruct((n, 128), jnp.float32),
            jax.ShapeDtypeStruct((n, 128), jnp.float32),
            jax.ShapeDtypeStruct((1, 128), jnp.float32),
            jax.ShapeDtypeStruct((1, 128), jnp.float32),
        ],
        compiler_params=pltpu.CompilerParams(
            dimension_semantics=("arbitrary",)),
    )(tok, gwu, gbu)


def _sc_dispatch(tok, pos_km, pt):
    n, h = tok.shape
    tpw = n // _NW
    mesh = plsc.VectorSubcoreMesh(core_axis_name="c", subcore_axis_name="s")

    @functools.partial(
        pl.kernel,
        out_type=jax.ShapeDtypeStruct((pt, h), jnp.float32),
        mesh=mesh,
        scratch_types=[
            pltpu.VMEM((tpw,), jnp.int32),
            pltpu.VMEM((tpw,), jnp.int32),
            pltpu.VMEM((tpw,), jnp.int32),
            pltpu.VMEM((tpw, h), jnp.float32),
            pltpu.SemaphoreType.DMA,
            pltpu.SemaphoreType.DMA,
            pltpu.SemaphoreType.DMA,
        ],
    )
    def k(tok_hbm, idx_hbm, x_hbm, idx0, idx1, idx2, rows_v, sem0, sem1, sem2):
        wid = lax.axis_index("s") * 2 + lax.axis_index("c")
        pltpu.sync_copy(tok_hbm.at[pl.ds(wid * tpw, tpw)], rows_v)
        idxv, sems = [idx0, idx1, idx2], [sem0, sem1, sem2]
        handles = []
        for kk in range(_K):
            pltpu.sync_copy(idx_hbm.at[kk * _NW + wid], idxv[kk])
            handles.append(
                pltpu.async_copy(rows_v, x_hbm.at[idxv[kk]], sems[kk]))
        for hd in handles:
            hd.wait()

    return k(tok, pos_km)


def _combine_kernel(ep_ref, pooled_ref, stats_ref, cnt_ref, fw1_ref, fb1_ref,
                    fw2_ref, uw1_ref, ub1_ref, uw2_ref, mb_ref,
                    out_ref, sc_ref):
    i = pl.program_id(0)
    pooled = pooled_ref[...] * (1.0 / 2048.0)
    t1 = jnp.maximum(
        jnp.dot(pooled, fw1_ref[...], preferred_element_type=jnp.float32)
        + fb1_ref[...], 0.0)
    phis = jax.nn.sigmoid(jnp.sum(t1 * fw2_ref[...], axis=1, keepdims=True)
                          + mb_ref[0, 0])
    scale = 1.0 - 0.8 * (phis > 0.7).astype(jnp.float32)
    out_ref[...] = jnp.sum(ep_ref[...], axis=0) * scale[0, 0]

    @pl.when(i == 0)
    def _scalars():
        um = pooled * scale
        u1 = jnp.maximum(
            jnp.dot(um, uw1_ref[...], preferred_element_type=jnp.float32)
            + ub1_ref[...], 0.0)
        unc = jax.nn.sigmoid(jnp.sum(u1 * uw2_ref[...], axis=1, keepdims=True)
                             + mb_ref[0, 1])
        pairs = jnp.float32(2048 * _K)
        conf_mean = stats_ref[0, 0] / pairs
        phim = stats_ref[0, 1] / pairs
        n1, n7, n11 = cnt_ref[0, 1], cnt_ref[0, 7], cnt_ref[0, 11]
        tri = jnp.where(n1 > 0,
                        stats_ref[0, 2] / (jnp.maximum(n1, 1.0) * 4.0), 0.0)
        drug = jnp.where(n7 > 0, stats_ref[0, 3] / jnp.maximum(n7, 1.0), 0.0)
        risk = jnp.where(n11 > 0,
                         stats_ref[0, 4] / (jnp.maximum(n11, 1.0) * 10.0), 0.0)
        ii = lax.broadcasted_iota(jnp.int32, (1, 128), 1)
        sc_ref[...] = (conf_mean * (ii == 0) + tri * (ii == 1)
                       + drug * (ii == 2) + risk * (ii == 3)
                       + phim * (ii == 4) + phis * (ii == 5)
                       + unc * (ii == 6) + scale * (ii == 7)
                       ).astype(jnp.float32)


def _run_combine(eo_pairs, pooled_s, stats, cnt, p, n, h):
    ts = 256
    h2, h4 = h // 2, h // 4
    mb = jnp.concatenate([p["fb2"].reshape(1, 1), p["uncb2"].reshape(1, 1),
                          jnp.zeros((1, 126), jnp.float32)], axis=1)
    return pl.pallas_call(
        _combine_kernel,
        grid=(n // ts,),
        in_specs=[
            pl.BlockSpec((_K, ts, h), lambda i: (0, i, 0)),
            pl.BlockSpec((1, h), lambda i: (0, 0)),
            pl.BlockSpec((1, 128), lambda i: (0, 0)),
            pl.BlockSpec((1, 128), lambda i: (0, 0)),
            pl.BlockSpec((h, h2), lambda i: (0, 0)),
            pl.BlockSpec((1, h2), lambda i: (0, 0)),
            pl.BlockSpec((1, h2), lambda i: (0, 0)),
            pl.BlockSpec((h, h4), lambda i: (0, 0)),
            pl.BlockSpec((1, h4), lambda i: (0, 0)),
            pl.BlockSpec((1, h4), lambda i: (0, 0)),
            pl.BlockSpec((1, 128), lambda i: (0, 0)),
        ],
        out_specs=[
            pl.BlockSpec((ts, h), lambda i: (i, 0)),
            pl.BlockSpec((1, 128), lambda i: (0, 0)),
        ],
        out_shape=[
            jax.ShapeDtypeStruct((n, h), jnp.float32),
            jax.ShapeDtypeStruct((1, 128), jnp.float32),
        ],
    )(eo_pairs.reshape(_K, n, h), pooled_s, stats, cnt,
      p["fW1"], p["fb1"].reshape(1, h2), p["fW2"].reshape(1, h2),
      p["uncW1"], p["uncb1"].reshape(1, h4), p["uncW2"].reshape(1, h4), mb)


def _ffn_kernel(te_ref, x_ref, w1_ref, w2_ref, aux_ref, phiw1_ref,
                tw1_ref, tb1_ref, twr_ref, tbr_ref,
                dw1_ref, db1_ref, dw2_ref, db2_ref, dw3_ref,
                rw1_ref, rb1_ref, rw2_ref, rb2_ref,
                we_ref,
                eow_ref, stats_ref, pooled_ref):
    i = pl.program_id(0)
    e = te_ref[0, i]
    lim = te_ref[1, i]

    @pl.when(i == 0)
    def _():
        stats_ref[...] = jnp.zeros_like(stats_ref)
        pooled_ref[...] = jnp.zeros_like(pooled_ref)

    @pl.when(lim > 0)
    def _body():
        aux = aux_ref[0]
        vcol = (lax.broadcasted_iota(jnp.int32, (_T, 1), 0)
                < lim).astype(jnp.float32)
        v = vcol[:, 0]
        wvv = we_ref[e] * v
        x = jnp.where(vcol > 0.0, x_ref[...], 0.0)
        h1a = jax.nn.gelu(
            jnp.dot(x, w1_ref[0, :, :1408], preferred_element_type=jnp.float32)
            + aux[:, 0:1408])
        eoa = jnp.dot(h1a, w2_ref[0, :1408, :],
                      preferred_element_type=jnp.float32)
        h1b = jax.nn.gelu(
            jnp.dot(x, w1_ref[0, :, 1408:], preferred_element_type=jnp.float32)
            + aux[:, 1408:2816])
        eob = jnp.dot(h1b, w2_ref[0, 1408:, :],
                      preferred_element_type=jnp.float32)
        eo = eoa + eob + aux[:, 2816:3840]
        eow = eo * wvv[:, None]
        eow_ref[...] = eow
        pooled_ref[...] += jnp.sum(eow, axis=0, keepdims=True)

        iota = lax.broadcasted_iota(jnp.int32, (1, 128), 1)
        conf = jax.nn.sigmoid(jnp.sum(eo * aux[:, 3840:4864], axis=1)
                              + aux_ref[0, 0, 5376])
        ph = jnp.maximum(
            jnp.dot(eo, phiw1_ref[0], preferred_element_type=jnp.float32)
            + aux[:, 4864:5120], 0.0)
        phi = jax.nn.sigmoid(jnp.sum(ph * aux[:, 5120:5376], axis=1)
                             + aux_ref[0, 0, 5377])
        stats_ref[...] += (jnp.where(iota == 0, jnp.sum(conf * v), 0.0)
                           + jnp.where(iota == 1, jnp.sum(phi * v), 0.0))

        @pl.when(e == 1)
        def _triage():
            t1 = jnp.maximum(
                jnp.dot(eo, tw1_ref[...], preferred_element_type=jnp.float32)
                + tb1_ref[...], 0.0)
            tl = (jnp.dot(t1, twr_ref[:, 0:128],
                          preferred_element_type=jnp.float32)
                  + tbr_ref[:, 0:128])
            t = jax.nn.softmax(tl, axis=-1)
            stats_ref[...] += jnp.where(iota == 2, jnp.sum(t * vcol), 0.0)

        @pl.when(e == 7)
        def _drug():
            d1 = jnp.maximum(
                jnp.dot(eo, dw1_ref[...], preferred_element_type=jnp.float32)
                + db1_ref[...], 0.0)
            d2 = jnp.maximum(
                jnp.dot(d1, dw2_ref[...], preferred_element_type=jnp.float32)
                + db2_ref[...], 0.0)
            d = jax.nn.sigmoid(jnp.sum(d2 * dw3_ref[...], axis=1)
                               + aux_ref[0, 0, 5378])
            stats_ref[...] += jnp.where(iota == 3, jnp.sum(d * v), 0.0)

        @pl.when(e == 11)
        def _risk():
            r1 = jnp.maximum(
                jnp.dot(eo, rw1_ref[...], preferred_element_type=jnp.float32)
                + rb1_ref[...], 0.0)
            r2 = jnp.maximum(
                jnp.dot(r1, rw2_ref[...], preferred_element_type=jnp.float32)
                + rb2_ref[...], 0.0)
            r = jax.nn.sigmoid(
                jnp.dot(r2, twr_ref[:, 128:256],
                        preferred_element_type=jnp.float32)
                + tbr_ref[:, 128:256])
            stats_ref[...] += jnp.where(iota == 4, jnp.sum(r * vcol), 0.0)


def _run_ffn(te2, X, we, p, NT, PT):
    E, H, I = p["W1"].shape
    H2, H4 = H // 2, H // 4
    f32 = jnp.float32

    neg = jnp.full((1, 124), -1e30, f32)
    twr = jnp.concatenate([p["tW2"], jnp.zeros((H2, 124), f32),
                           p["rW3"], jnp.zeros((H2, 118), f32)], axis=1)
    tbr = jnp.concatenate([p["tb2"].reshape(1, 4), neg,
                           p["rb3"].reshape(1, 10), neg[:, :118]], axis=1)
    aux = jnp.concatenate([
        p["b1"], p["b2"], p["confW"], p["phib1"], p["phiW2"],
        p["confb"][:, None], p["phib2"][:, None],
        jnp.broadcast_to(p["db3"], (E,))[:, None],
        jnp.zeros((E, 125), f32)], axis=1)[:, None, :]

    grid_spec = pltpu.PrefetchScalarGridSpec(
        num_scalar_prefetch=1,
        grid=(NT,),
        in_specs=[
            pl.BlockSpec((_T, H), lambda i, te: (i, 0)),
            pl.BlockSpec((1, H, I), lambda i, te: (te[0, i], 0, 0)),
            pl.BlockSpec((1, I, H), lambda i, te: (te[0, i], 0, 0)),
            pl.BlockSpec((1, 1, 5504), lambda i, te: (te[0, i], 0, 0)),
            pl.BlockSpec((1, H, H4), lambda i, te: (te[0, i], 0, 0)),
            pl.BlockSpec((H, H2), lambda i, te: (0, 0)),
            pl.BlockSpec((1, H2), lambda i, te: (0, 0)),
            pl.BlockSpec((H2, 256), lambda i, te: (0, 0)),
            pl.BlockSpec((1, 256), lambda i, te: (0, 0)),
            pl.BlockSpec((H, H2), lambda i, te: (0, 0)),
            pl.BlockSpec((1, H2), lambda i, te: (0, 0)),
            pl.BlockSpec((H2, H4), lambda i, te: (0, 0)),
            pl.BlockSpec((1, H4), lambda i, te: (0, 0)),
            pl.BlockSpec((1, H4), lambda i, te: (0, 0)),
            pl.BlockSpec((H, H), lambda i, te: (0, 0)),
            pl.BlockSpec((1, H), lambda i, te: (0, 0)),
            pl.BlockSpec((H, H2), lambda i, te: (0, 0)),
            pl.BlockSpec((1, H2), lambda i, te: (0, 0)),
            pl.BlockSpec(memory_space=pltpu.SMEM),
        ],
        out_specs=[
            pl.BlockSpec((_T, H), lambda i, te: (i, 0)),
            pl.BlockSpec((1, 128), lambda i, te: (0, 0)),
            pl.BlockSpec((1, H), lambda i, te: (0, 0)),
        ],
    )
    eow, stats, pooled = pl.pallas_call(
        _ffn_kernel,
        grid_spec=grid_spec,
        out_shape=[
            jax.ShapeDtypeStruct((PT, H), f32),
            jax.ShapeDtypeStruct((1, 128), f32),
            jax.ShapeDtypeStruct((1, H), f32),
        ],
        compiler_params=pltpu.CompilerParams(
            dimension_semantics=("arbitrary",),
            vmem_limit_bytes=100 * 1024 * 1024),
    )(te2, X, p["W1"], p["W2"], aux, p["phiW1"],
      p["tW1"], p["tb1"].reshape(1, H2), twr, tbr,
      p["dW1"], p["db1"].reshape(1, H2), p["dW2"], p["db2"].reshape(1, H4),
      p["dW3"].reshape(1, H4),
      p["rW1"], p["rb1"].reshape(1, H), p["rW2"], p["rb2"].reshape(1, H2),
      we)
    return eow, stats, pooled


def kernel(hidden_states, params):
    p = params
    b, s, h = hidden_states.shape
    n = b * s
    E = p["gW"].shape[1]
    f32 = jnp.float32
    tok = hidden_states.reshape(n, h)

    gwu = jnp.concatenate([p["gW"], p["uW"][:, None],
                           jnp.zeros((h, 115), f32)], axis=1)
    gbu = jnp.concatenate([p["gb"], p["ub"].reshape(1),
                           jnp.zeros((115,), f32)]).reshape(1, 128)
    probs128, misc, cnt128, ww128 = _run_router(tok, gwu, gbu, n, h)

    specialty_probs = probs128[:, :E].reshape(b, s, E)
    urgency = misc[:, 0].reshape(b, s)
    topi = misc[:, 4:4 + _K].astype(jnp.int32)
    rank_f = misc[:, 7:7 + _K]
    P = n * _K
    e_flat = topi.reshape(-1)
    cnt_f = cnt128[0, :E]
    counts = cnt_f.astype(jnp.int32)
    wsum = ww128[0, :E]
    w_e = jnp.where(counts > 0, wsum / jnp.maximum(cnt_f, 1.0), 0.0)

    offs = jnp.cumsum(counts) - counts
    pcounts = ((counts + _T - 1) // _T) * _T
    cum_p = jnp.cumsum(pcounts)
    poffs = cum_p - pcounts
    NT = P // _T + E
    PT = NT * _T
    tiles = jnp.arange(NT, dtype=jnp.int32)
    cum_pt = cum_p // _T
    tile_e = jnp.clip(jnp.sum((tiles[:, None] >= cum_pt[None, :]),
                              axis=1).astype(jnp.int32), 0, E - 1)
    lim = jnp.clip(counts[tile_e] - (tiles * _T - poffs[tile_e]), 0, _T)
    te2 = jnp.stack([tile_e, lim], axis=0)

    pos = poffs[e_flat] + rank_f.reshape(-1).astype(jnp.int32)
    pos_km = pos.reshape(n, _K).T

    X = _sc_dispatch(tok, pos_km.reshape(_K * _NW, n // _NW), PT)
    eow, stats, pooled_s = _run_ffn(te2, X, w_e, p, NT, PT)

    eo_pairs = _sc_row_gather(eow, pos_km.reshape(-1))
    out_rows, ep = _run_combine(eo_pairs, pooled_s, stats, cnt128, p, n, h)
    conf_mean = ep[0, 0]
    triage_mean = ep[0, 1]
    drug_mean = ep[0, 2]
    risk_mean = ep[0, 3]
    phi_prob_mean = ep[0, 4]
    phi_score = ep[0, 5:6]
    uncertainty = ep[0, 6:7]
    output = out_rows.reshape(b, s, h)

    return (output, specialty_probs, urgency, topi.reshape(b, s, _K),
            conf_mean, triage_mean, drug_mean, risk_mean, phi_prob_mean,
            phi_score, uncertainty)


# speedup vs baseline: 1.5411x; 1.5411x over previous
import functools

import jax
import jax.numpy as jnp
from jax import lax
from jax.experimental import pallas as pl
from jax.experimental.pallas import tpu as pltpu
from jax.experimental.pallas import tpu_sc as plsc

_T = 192
_K = 3
_NW = 32


def _sc_row_gather(table, idx):
    m = idx.shape[0]
    h = table.shape[1]
    per_w = m // _NW
    ch = 48
    nch = per_w // ch
    assert per_w % ch == 0
    mesh = plsc.VectorSubcoreMesh(core_axis_name="c", subcore_axis_name="s")

    @functools.partial(
        pl.kernel,
        out_type=jax.ShapeDtypeStruct((m, h), jnp.float32),
        mesh=mesh,
        scratch_types=[
            pltpu.VMEM((ch,), jnp.int32),
            pltpu.VMEM((ch,), jnp.int32),
            pltpu.VMEM((ch, h), jnp.float32),
            pltpu.VMEM((ch, h), jnp.float32),
            pltpu.SemaphoreType.DMA,
            pltpu.SemaphoreType.DMA,
        ],
    )
    def k(table_hbm, idx_hbm, out_hbm, idx0, idx1, rows0, rows1, sem0, sem1):
        wid = lax.axis_index("s") * 2 + lax.axis_index("c")
        base = wid * per_w
        idxv, rowsv, sems = [idx0, idx1], [rows0, rows1], [sem0, sem1]
        handles = [None, None]
        pltpu.sync_copy(idx_hbm.at[pl.ds(base, ch)], idx0)
        handles[0] = pltpu.async_copy(table_hbm.at[idx0], rows0, sem0)
        for c in range(nch):
            j, jn = c % 2, (c + 1) % 2
            if c + 1 < nch:
                off = base + (c + 1) * ch
                pltpu.sync_copy(idx_hbm.at[pl.ds(off, ch)], idxv[jn])
                handles[jn] = pltpu.async_copy(
                    table_hbm.at[idxv[jn]], rowsv[jn], sems[jn])
            handles[j].wait()
            pltpu.sync_copy(rowsv[j], out_hbm.at[pl.ds(base + c * ch, ch)])

    return k(table, idx)


def _router_kernel(x_ref, gwu_ref, gbu_ref, probs_ref, misc_ref,
                   cnt_ref, ww_ref):
    i = pl.program_id(0)

    @pl.when(i == 0)
    def _():
        cnt_ref[...] = jnp.zeros_like(cnt_ref)
        ww_ref[...] = jnp.zeros_like(ww_ref)

    x = x_ref[...]
    la = jnp.dot(x, gwu_ref[...], preferred_element_type=jnp.float32) \
        + gbu_ref[...]
    ii = lax.broadcasted_iota(jnp.int32, la.shape, 1)
    ml = jnp.where(ii < 12, la, -1e30)
    probs = jax.nn.softmax(ml, axis=-1)
    probs_ref[...] = probs
    urg = jax.nn.sigmoid(jnp.sum(jnp.where(ii == 12, la, 0.0), axis=1,
                                 keepdims=True))

    pr = probs
    ams, vs, hs = [], [], []
    for _ in range(_K):
        m = jnp.max(pr, axis=1, keepdims=True)
        am = jnp.min(jnp.where(pr == m, ii, 128), axis=1, keepdims=True)
        h = (ii == am).astype(jnp.float32)
        pr = jnp.where(ii == am, -1.0, pr)
        ams.append(am)
        vs.append(m)
        hs.append(h)
    e1 = jnp.exp(vs[1] - vs[0])
    e2 = jnp.exp(vs[2] - vs[0])
    den = 1.0 + e1 + e2
    ews = [1.0 / den, e1 / den, e2 / den]

    c = hs[0] + hs[1] + hs[2]
    ri = lax.broadcasted_iota(jnp.int32, (c.shape[0], c.shape[0]), 0)
    ci = lax.broadcasted_iota(jnp.int32, (c.shape[0], c.shape[0]), 1)
    ltri = (ri > ci).astype(jnp.float32)
    base = jnp.dot(ltri, c, preferred_element_type=jnp.float32)
    carry = cnt_ref[...]
    misc = urg * (ii == 0).astype(jnp.float32)
    for k in range(_K):
        rank = jnp.sum((base + carry) * hs[k], axis=1, keepdims=True)
        misc = misc + ews[k] * (ii == 1 + k).astype(jnp.float32)
        misc = misc + ams[k].astype(jnp.float32) * (ii == 4 + k).astype(jnp.float32)
        misc = misc + rank * (ii == 7 + k).astype(jnp.float32)
    misc_ref[...] = misc
    cnt_ref[...] += jnp.sum(c, axis=0, keepdims=True)
    ww_ref[...] += jnp.sum(ews[0] * hs[0] + ews[1] * hs[1] + ews[2] * hs[2],
                           axis=0, keepdims=True)


def _run_router(tok, gwu, gbu, n, h):
    ts = 512
    return pl.pallas_call(
        _router_kernel,
        grid=(n // ts,),
        in_specs=[
            pl.BlockSpec((ts, h), lambda i: (i, 0)),
            pl.BlockSpec((h, 128), lambda i: (0, 0)),
            pl.BlockSpec((1, 128), lambda i: (0, 0)),
        ],
        out_specs=[
            pl.BlockSpec((ts, 128), lambda i: (i, 0)),
            pl.BlockSpec((ts, 128), lambda i: (i, 0)),
            pl.BlockSpec((1, 128), lambda i: (0, 0)),
            pl.BlockSpec((1, 128), lambda i: (0, 0)),
        ],
        out_shape=[
            jax.ShapeDtypeStruct((n, 128), jnp.float32),
            jax.ShapeDtypeStruct((n, 128), jnp.float32),
            jax.ShapeDtypeStruct((1, 128), jnp.float32),
            jax.ShapeDtypeStruct((1, 128), jnp.float32),
        ],
        compiler_params=pltpu.CompilerParams(
            dimension_semantics=("arbitrary",)),
    )(tok, gwu, gbu)


def _sc_dispatch(tok, pos_km, pt):
    n, h = tok.shape
    tpw = n // _NW
    mesh = plsc.VectorSubcoreMesh(core_axis_name="c", subcore_axis_name="s")

    @functools.partial(
        pl.kernel,
        out_type=jax.ShapeDtypeStruct((pt, h), jnp.float32),
        mesh=mesh,
        scratch_types=[
            pltpu.VMEM((tpw,), jnp.int32),
            pltpu.VMEM((tpw,), jnp.int32),
            pltpu.VMEM((tpw,), jnp.int32),
            pltpu.VMEM((tpw, h), jnp.float32),
            pltpu.SemaphoreType.DMA,
            pltpu.SemaphoreType.DMA,
            pltpu.SemaphoreType.DMA,
        ],
    )
    def k(tok_hbm, idx_hbm, x_hbm, idx0, idx1, idx2, rows_v, sem0, sem1, sem2):
        wid = lax.axis_index("s") * 2 + lax.axis_index("c")
        pltpu.sync_copy(tok_hbm.at[pl.ds(wid * tpw, tpw)], rows_v)
        idxv, sems = [idx0, idx1, idx2], [sem0, sem1, sem2]
        handles = []
        for kk in range(_K):
            pltpu.sync_copy(idx_hbm.at[kk * _NW + wid], idxv[kk])
            handles.append(
                pltpu.async_copy(rows_v, x_hbm.at[idxv[kk]], sems[kk]))
        for hd in handles:
            hd.wait()

    return k(tok, pos_km)


def _combine_kernel(ep_ref, pooled_ref, stats_ref, cnt_ref, fw1_ref, fb1_ref,
                    fw2_ref, uw1_ref, ub1_ref, uw2_ref, mb_ref,
                    out_ref, sc_ref):
    i = pl.program_id(0)
    pooled = pooled_ref[...] * (1.0 / 2048.0)
    t1 = jnp.maximum(
        jnp.dot(pooled, fw1_ref[...], preferred_element_type=jnp.float32)
        + fb1_ref[...], 0.0)
    phis = jax.nn.sigmoid(jnp.sum(t1 * fw2_ref[...], axis=1, keepdims=True)
                          + mb_ref[0, 0])
    scale = 1.0 - 0.8 * (phis > 0.7).astype(jnp.float32)
    out_ref[...] = jnp.sum(ep_ref[...], axis=0) * scale[0, 0]

    @pl.when(i == 0)
    def _scalars():
        um = pooled * scale
        u1 = jnp.maximum(
            jnp.dot(um, uw1_ref[...], preferred_element_type=jnp.float32)
            + ub1_ref[...], 0.0)
        unc = jax.nn.sigmoid(jnp.sum(u1 * uw2_ref[...], axis=1, keepdims=True)
                             + mb_ref[0, 1])
        pairs = jnp.float32(2048 * _K)
        conf_mean = stats_ref[0, 0] / pairs
        phim = stats_ref[0, 1] / pairs
        n1, n7, n11 = cnt_ref[0, 1], cnt_ref[0, 7], cnt_ref[0, 11]
        tri = jnp.where(n1 > 0,
                        stats_ref[0, 2] / (jnp.maximum(n1, 1.0) * 4.0), 0.0)
        drug = jnp.where(n7 > 0, stats_ref[0, 3] / jnp.maximum(n7, 1.0), 0.0)
        risk = jnp.where(n11 > 0,
                         stats_ref[0, 4] / (jnp.maximum(n11, 1.0) * 10.0), 0.0)
        ii = lax.broadcasted_iota(jnp.int32, (1, 128), 1)
        sc_ref[...] = (conf_mean * (ii == 0) + tri * (ii == 1)
                       + drug * (ii == 2) + risk * (ii == 3)
                       + phim * (ii == 4) + phis * (ii == 5)
                       + unc * (ii == 6) + scale * (ii == 7)
                       ).astype(jnp.float32)


def _run_combine(eo_pairs, pooled_s, stats, cnt, p, n, h):
    ts = 256
    h2, h4 = h // 2, h // 4
    mb = jnp.concatenate([p["fb2"].reshape(1, 1), p["uncb2"].reshape(1, 1),
                          jnp.zeros((1, 126), jnp.float32)], axis=1)
    return pl.pallas_call(
        _combine_kernel,
        grid=(n // ts,),
        in_specs=[
            pl.BlockSpec((_K, ts, h), lambda i: (0, i, 0)),
            pl.BlockSpec((1, h), lambda i: (0, 0)),
            pl.BlockSpec((1, 128), lambda i: (0, 0)),
            pl.BlockSpec((1, 128), lambda i: (0, 0)),
            pl.BlockSpec((h, h2), lambda i: (0, 0)),
            pl.BlockSpec((1, h2), lambda i: (0, 0)),
            pl.BlockSpec((1, h2), lambda i: (0, 0)),
            pl.BlockSpec((h, h4), lambda i: (0, 0)),
            pl.BlockSpec((1, h4), lambda i: (0, 0)),
            pl.BlockSpec((1, h4), lambda i: (0, 0)),
            pl.BlockSpec((1, 128), lambda i: (0, 0)),
        ],
        out_specs=[
            pl.BlockSpec((ts, h), lambda i: (i, 0)),
            pl.BlockSpec((1, 128), lambda i: (0, 0)),
        ],
        out_shape=[
            jax.ShapeDtypeStruct((n, h), jnp.float32),
            jax.ShapeDtypeStruct((1, 128), jnp.float32),
        ],
    )(eo_pairs.reshape(_K, n, h), pooled_s, stats, cnt,
      p["fW1"], p["fb1"].reshape(1, h2), p["fW2"].reshape(1, h2),
      p["uncW1"], p["uncb1"].reshape(1, h4), p["uncW2"].reshape(1, h4), mb)


def _ffn_kernel(te_ref, x_ref, w1_ref, w2_ref, aux_ref, phiw1_ref,
                tw1_ref, tb1_ref, twr_ref, tbr_ref,
                dw1_ref, db1_ref, dw2_ref, db2_ref, dw3_ref,
                rw1_ref, rb1_ref, rw2_ref, rb2_ref,
                we_ref,
                eow_ref, stats_ref, pooled_ref):
    i = pl.program_id(0)
    e = te_ref[0, i]
    lim = te_ref[1, i]

    @pl.when(i == 0)
    def _():
        stats_ref[...] = jnp.zeros_like(stats_ref)
        pooled_ref[...] = jnp.zeros_like(pooled_ref)

    @pl.when(lim > 0)
    def _body():
        aux = aux_ref[0]
        vcol = (lax.broadcasted_iota(jnp.int32, (_T, 1), 0)
                < lim).astype(jnp.float32)
        v = vcol[:, 0]
        wvv = we_ref[e] * v
        x = jnp.where(vcol > 0.0, x_ref[...], 0.0)
        h1a = jax.nn.gelu(
            jnp.dot(x, w1_ref[0, :, :1408], preferred_element_type=jnp.float32)
            + aux[:, 0:1408])
        eoa = jnp.dot(h1a, w2_ref[0, :1408, :],
                      preferred_element_type=jnp.float32)
        h1b = jax.nn.gelu(
            jnp.dot(x, w1_ref[0, :, 1408:], preferred_element_type=jnp.float32)
            + aux[:, 1408:2816])
        eob = jnp.dot(h1b, w2_ref[0, 1408:, :],
                      preferred_element_type=jnp.float32)
        eo = eoa + eob + aux[:, 2816:3840]
        eow = eo * wvv[:, None]
        eow_ref[...] = eow
        pooled_ref[...] += jnp.sum(eow, axis=0, keepdims=True)

        iota = lax.broadcasted_iota(jnp.int32, (1, 128), 1)
        conf = jax.nn.sigmoid(jnp.sum(eo * aux[:, 3840:4864], axis=1)
                              + aux_ref[0, 0, 5376])
        ph = jnp.maximum(
            jnp.dot(eo, phiw1_ref[0], preferred_element_type=jnp.float32)
            + aux[:, 4864:5120], 0.0)
        phi = jax.nn.sigmoid(jnp.sum(ph * aux[:, 5120:5376], axis=1)
                             + aux_ref[0, 0, 5377])
        stats_ref[...] += (jnp.where(iota == 0, jnp.sum(conf * v), 0.0)
                           + jnp.where(iota == 1, jnp.sum(phi * v), 0.0))

        @pl.when(e == 1)
        def _triage():
            t1 = jnp.maximum(
                jnp.dot(eo, tw1_ref[...], preferred_element_type=jnp.float32)
                + tb1_ref[...], 0.0)
            tl = (jnp.dot(t1, twr_ref[:, 0:128],
                          preferred_element_type=jnp.float32)
                  + tbr_ref[:, 0:128])
            t = jax.nn.softmax(tl, axis=-1)
            stats_ref[...] += jnp.where(iota == 2, jnp.sum(t * vcol), 0.0)

        @pl.when(e == 7)
        def _drug():
            d1 = jnp.maximum(
                jnp.dot(eo, dw1_ref[...], preferred_element_type=jnp.float32)
                + db1_ref[...], 0.0)
            d2 = jnp.maximum(
                jnp.dot(d1, dw2_ref[...], preferred_element_type=jnp.float32)
                + db2_ref[...], 0.0)
            d = jax.nn.sigmoid(jnp.sum(d2 * dw3_ref[...], axis=1)
                               + aux_ref[0, 0, 5378])
            stats_ref[...] += jnp.where(iota == 3, jnp.sum(d * v), 0.0)

        @pl.when(e == 11)
        def _risk():
            r1 = jnp.maximum(
                jnp.dot(eo, rw1_ref[...], preferred_element_type=jnp.float32)
                + rb1_ref[...], 0.0)
            r2 = jnp.maximum(
                jnp.dot(r1, rw2_ref[...], preferred_element_type=jnp.float32)
                + rb2_ref[...], 0.0)
            r = jax.nn.sigmoid(
                jnp.dot(r2, twr_ref[:, 128:256],
                        preferred_element_type=jnp.float32)
                + tbr_ref[:, 128:256])
            stats_ref[...] += jnp.where(iota == 4, jnp.sum(r * vcol), 0.0)


def _run_ffn(te2, X, we, p, NT, PT):
    E, H, I = p["W1"].shape
    H2, H4 = H // 2, H // 4
    f32 = jnp.float32

    neg = jnp.full((1, 124), -1e30, f32)
    twr = jnp.concatenate([p["tW2"], jnp.zeros((H2, 124), f32),
                           p["rW3"], jnp.zeros((H2, 118), f32)], axis=1)
    tbr = jnp.concatenate([p["tb2"].reshape(1, 4), neg,
                           p["rb3"].reshape(1, 10), neg[:, :118]], axis=1)
    aux = jnp.concatenate([
        p["b1"], p["b2"], p["confW"], p["phib1"], p["phiW2"],
        p["confb"][:, None], p["phib2"][:, None],
        jnp.broadcast_to(p["db3"], (E,))[:, None],
        jnp.zeros((E, 125), f32)], axis=1)[:, None, :]

    grid_spec = pltpu.PrefetchScalarGridSpec(
        num_scalar_prefetch=1,
        grid=(NT,),
        in_specs=[
            pl.BlockSpec((_T, H), lambda i, te: (i, 0)),
            pl.BlockSpec((1, H, I), lambda i, te: (te[0, i], 0, 0)),
            pl.BlockSpec((1, I, H), lambda i, te: (te[0, i], 0, 0)),
            pl.BlockSpec((1, 1, 5504), lambda i, te: (te[0, i], 0, 0)),
            pl.BlockSpec((1, H, H4), lambda i, te: (te[0, i], 0, 0)),
            pl.BlockSpec((H, H2), lambda i, te: (0, 0)),
            pl.BlockSpec((1, H2), lambda i, te: (0, 0)),
            pl.BlockSpec((H2, 256), lambda i, te: (0, 0)),
            pl.BlockSpec((1, 256), lambda i, te: (0, 0)),
            pl.BlockSpec((H, H2), lambda i, te: (0, 0)),
            pl.BlockSpec((1, H2), lambda i, te: (0, 0)),
            pl.BlockSpec((H2, H4), lambda i, te: (0, 0)),
            pl.BlockSpec((1, H4), lambda i, te: (0, 0)),
            pl.BlockSpec((1, H4), lambda i, te: (0, 0)),
            pl.BlockSpec((H, H), lambda i, te: (0, 0)),
            pl.BlockSpec((1, H), lambda i, te: (0, 0)),
            pl.BlockSpec((H, H2), lambda i, te: (0, 0)),
            pl.BlockSpec((1, H2), lambda i, te: (0, 0)),
            pl.BlockSpec(memory_space=pltpu.SMEM),
        ],
        out_specs=[
            pl.BlockSpec((_T, H), lambda i, te: (i, 0)),
            pl.BlockSpec((1, 128), lambda i, te: (0, 0)),
            pl.BlockSpec((1, H), lambda i, te: (0, 0)),
        ],
    )
    eow, stats, pooled = pl.pallas_call(
        _ffn_kernel,
        grid_spec=grid_spec,
        out_shape=[
            jax.ShapeDtypeStruct((PT, H), f32),
            jax.ShapeDtypeStruct((1, 128), f32),
            jax.ShapeDtypeStruct((1, H), f32),
        ],
        compiler_params=pltpu.CompilerParams(
            dimension_semantics=("arbitrary",),
            vmem_limit_bytes=100 * 1024 * 1024),
    )(te2, X, p["W1"], p["W2"], aux, p["phiW1"],
      p["tW1"], p["tb1"].reshape(1, H2), twr, tbr,
      p["dW1"], p["db1"].reshape(1, H2), p["dW2"], p["db2"].reshape(1, H4),
      p["dW3"].reshape(1, H4),
      p["rW1"], p["rb1"].reshape(1, H), p["rW2"], p["rb2"].reshape(1, H2),
      we)
    return eow, stats, pooled


def kernel(hidden_states, params):
    p = params
    b, s, h = hidden_states.shape
    n = b * s
    E = p["gW"].shape[1]
    f32 = jnp.float32
    tok = hidden_states.reshape(n, h)

    gwu = jnp.concatenate([p["gW"], p["uW"][:, None],
                           jnp.zeros((h, 115), f32)], axis=1)
    gbu = jnp.concatenate([p["gb"], p["ub"].reshape(1),
                           jnp.zeros((115,), f32)]).reshape(1, 128)
    probs128, misc, cnt128, ww128 = _run_router(tok, gwu, gbu, n, h)

    specialty_probs = probs128[:, :E].reshape(b, s, E)
    urgency = misc[:, 0].reshape(b, s)
    topi = misc[:, 4:4 + _K].astype(jnp.int32)
    rank_f = misc[:, 7:7 + _K]
    P = n * _K
    e_flat = topi.reshape(-1)
    cnt_f = cnt128[0, :E]
    counts = cnt_f.astype(jnp.int32)
    wsum = ww128[0, :E]
    w_e = jnp.where(counts > 0, wsum / jnp.maximum(cnt_f, 1.0), 0.0)

    offs = jnp.cumsum(counts) - counts
    pcounts = ((counts + _T - 1) // _T) * _T
    cum_p = jnp.cumsum(pcounts)
    poffs = cum_p - pcounts
    NT = P // _T + E
    PT = NT * _T
    tiles = jnp.arange(NT, dtype=jnp.int32)
    cum_pt = cum_p // _T
    tile_e = jnp.clip(jnp.sum((tiles[:, None] >= cum_pt[None, :]),
                              axis=1).astype(jnp.int32), 0, E - 1)
    lim = jnp.clip(counts[tile_e] - (tiles * _T - poffs[tile_e]), 0, _T)
    te2 = jnp.stack([tile_e, lim], axis=0)

    pos = poffs[e_flat] + rank_f.reshape(-1).astype(jnp.int32)
    pos_km = pos.reshape(n, _K).T

    X = _sc_dispatch(tok, pos_km.reshape(_K * _NW, n // _NW), PT)
    eow, stats, pooled_s = _run_ffn(te2, X, w_e, p, NT, PT)

    eo_pairs = _sc_row_gather(eow, pos_km.reshape(-1))
    out_rows, ep = _run_combine(eo_pairs, pooled_s, stats, cnt128, p, n, h)
    conf_mean = ep[0, 0]
    triage_mean = ep[0, 1]
    drug_mean = ep[0, 2]
    risk_mean = ep[0, 3]
    phi_prob_mean = ep[0, 4]
    phi_score = ep[0, 5:6]
    uncertainty = ep[0, 6:7]
    output = out_rows.reshape(b, s, h)

    return (output, specialty_probs, urgency, topi.reshape(b, s, _K),
            conf_mean, triage_mean, drug_mean, risk_mean, phi_prob_mean,
            phi_score, uncertainty)
